# baseline (speedup 1.0000x reference)
import functools

import jax
import jax.numpy as jnp
from jax import lax
from jax.experimental import pallas as pl
from jax.experimental.pallas import tpu as pltpu
from jax.experimental.pallas import tpu_sc as plsc

NC = 2
NS = 16
NW = NC * NS
CH = 80
BN = 1000

_mesh = plsc.VectorSubcoreMesh(core_axis_name="c", subcore_axis_name="s")


def _sc_count(dst, et, N, R, E):
    RN = R * N
    EPW = E // NW
    NCH = EPW // CH
    ZB = RN // NS

    @functools.partial(
        pl.kernel,
        out_type=jax.ShapeDtypeStruct((NC * RN,), jnp.float32),
        mesh=_mesh,
        compiler_params=pltpu.CompilerParams(use_tc_tiling_on_sc=False),
        scratch_types=[
            pltpu.VMEM_SHARED((RN,), jnp.float32),
            pltpu.VMEM((CH,), jnp.int32),
            pltpu.VMEM((CH,), jnp.int32),
            pltpu.VMEM((CH,), jnp.int32),
            pltpu.VMEM((CH,), jnp.int32),
            pltpu.VMEM((CH,), jnp.int32),
            pltpu.VMEM((CH,), jnp.int32),
            pltpu.VMEM((CH,), jnp.float32),
            pltpu.VMEM((CH,), jnp.float32),
            pltpu.VMEM((ZB,), jnp.float32),
            pltpu.SemaphoreType.DMA,
            pltpu.SemaphoreType.DMA,
            pltpu.SemaphoreType.DMA,
        ],
    )
    def k(dst_h, et_h, cnt_o, cnt_sh,
          d_v0, t_v0, i_v0, d_v1, t_v1, i_v1, ones_v, zc_v, z_v,
          sem_e, sem_c0, sem_c1):
        cid = lax.axis_index("c")
        sid = lax.axis_index("s")
        wid = sid * NC + cid
        dv = (d_v0, d_v1)
        tv = (t_v0, t_v1)
        iv = (i_v0, i_v1)
        semc = (sem_c0, sem_c1)

        def zlp(j, c):
            z_v[pl.ds(j * 16, 16)] = jnp.zeros((16,), jnp.float32)
            return c

        lax.fori_loop(0, ZB // 16, zlp, 0)
        pltpu.sync_copy(z_v, cnt_sh.at[pl.ds(sid * ZB, ZB)])
        for j in range(CH // 16):
            s_ = pl.ds(j * 16, 16)
            ones_v[s_] = jnp.ones((16,), jnp.float32)
            zc_v[s_] = jnp.zeros((16,), jnp.float32)
            i_v0[s_] = jnp.zeros((16,), jnp.int32)
            i_v1[s_] = jnp.zeros((16,), jnp.int32)
        pltpu.async_copy(zc_v, cnt_sh.at[i_v0], add=True, sem=sem_c0)
        pltpu.async_copy(zc_v, cnt_sh.at[i_v1], add=True, sem=sem_c1)
        plsc.subcore_barrier()

        def load_edges(i, p):
            b = pl.multiple_of(wid * EPW + i * CH, 8)
            pltpu.async_copy(dst_h.at[pl.ds(b, CH)], dv[p], sem_e)
            pltpu.async_copy(et_h.at[pl.ds(b, CH)], tv[p], sem_e)

        def wait_edges(p):
            pltpu.make_async_copy(dst_h.at[pl.ds(0, CH)], dv[p], sem_e).wait()
            pltpu.make_async_copy(et_h.at[pl.ds(0, CH)], tv[p], sem_e).wait()

        def step(i, p, po):
            wait_edges(p)
            pltpu.make_async_copy(ones_v, cnt_sh.at[iv[p]], semc[p]).wait()
            for j in range(CH // 16):
                s_ = pl.ds(j * 16, 16)
                iv[p][s_] = dv[p][s_] * R + tv[p][s_]

            @pl.when(i + 1 < NCH)
            def _():
                load_edges(i + 1, po)

            pltpu.async_copy(ones_v, cnt_sh.at[iv[p]], add=True, sem=semc[p])

        load_edges(0, 0)

        def body(kk, c):
            step(2 * kk, 0, 1)
            step(2 * kk + 1, 1, 0)
            return c

        lax.fori_loop(0, (NCH - 1) // 2, body, 0)
        step(NCH - 1, 0, 1)
        pltpu.make_async_copy(ones_v, cnt_sh.at[i_v0], sem_c0).wait()
        pltpu.make_async_copy(ones_v, cnt_sh.at[i_v1], sem_c1).wait()
        plsc.subcore_barrier()
        pltpu.sync_copy(cnt_sh.at[pl.ds(sid * ZB, ZB)], z_v)
        pltpu.sync_copy(z_v, cnt_o.at[pl.ds(cid * RN + sid * ZB, ZB)])

    return k(dst, et)


def _sc_pass(src, dst, et, yf, w16, N, R, E, D):
    EPW = E // NW
    NCH = EPW // CH
    assert NCH % 2 == 1
    ZR = 80
    NZC = N // ZR
    NP = (NZC + NS - 1) // NS

    @functools.partial(
        pl.kernel,
        out_type=jax.ShapeDtypeStruct((NC, N, D), jnp.float32),
        mesh=_mesh,
        compiler_params=pltpu.CompilerParams(use_tc_tiling_on_sc=False),
        scratch_types=[
            pltpu.VMEM_SHARED((N, D), jnp.float32),
            pltpu.VMEM((CH,), jnp.int32),
            pltpu.VMEM((CH,), jnp.int32),
            pltpu.VMEM((CH,), jnp.int32),
            pltpu.VMEM((CH,), jnp.int32),
            pltpu.VMEM((CH,), jnp.int32),
            pltpu.VMEM((CH,), jnp.int32),
            pltpu.VMEM((CH,), jnp.int32),
            pltpu.VMEM((CH,), jnp.int32),
            pltpu.VMEM((CH,), jnp.int32),
            pltpu.VMEM((CH,), jnp.int32),
            pltpu.VMEM((CH, 16), jnp.float32),
            pltpu.VMEM((CH, 16), jnp.float32),
            pltpu.VMEM((CH, D), jnp.float32),
            pltpu.VMEM((CH, D), jnp.float32),
            pltpu.VMEM((ZR, D), jnp.float32),
            pltpu.VMEM((CH,), jnp.int32),
            pltpu.VMEM((CH,), jnp.int32),
            pltpu.VMEM((CH,), jnp.int32),
            pltpu.SemaphoreType.DMA,
            pltpu.SemaphoreType.DMA,
            pltpu.SemaphoreType.DMA,
            pltpu.SemaphoreType.DMA,
        ],
    )
    def k(src_h, dst_h, et_h, y_h, w16_h, acc_o, acc_sh,
          s_v0, d_v0, t_v0, g_v0, q_v0,
          s_v1, d_v1, t_v1, g_v1, q_v1,
          w_v0, w_v1, rows_v0, rows_v1, zr_v, si_v0, si_v1, izero,
          sem_e, sem_g, sem_sc0, sem_sc1):
        cid = lax.axis_index("c")
        sid = lax.axis_index("s")
        wid = sid * NC + cid
        sv = (s_v0, s_v1)
        dv = (d_v0, d_v1)
        tv = (t_v0, t_v1)
        gv = (g_v0, g_v1)
        qv = (q_v0, q_v1)
        wv = (w_v0, w_v1)
        rowsv = (rows_v0, rows_v1)
        siv = (si_v0, si_v1)
        semsc = (sem_sc0, sem_sc1)

        def zlp(i, c):
            for j in range(D // 16):
                zr_v[i, pl.ds(j * 16, 16)] = jnp.zeros((16,), jnp.float32)
            return c

        lax.fori_loop(0, ZR, zlp, 0)
        for p in range(NP):
            cidx = sid + p * NS

            @pl.when(cidx < NZC)
            def _():
                pltpu.sync_copy(zr_v, acc_sh.at[pl.ds(cidx * ZR, ZR)])

        plsc.subcore_barrier()

        def load_edges(i, p):
            b = pl.multiple_of(wid * EPW + i * CH, 8)
            pltpu.async_copy(src_h.at[pl.ds(b, CH)], sv[p], sem_e)
            pltpu.async_copy(dst_h.at[pl.ds(b, CH)], dv[p], sem_e)
            pltpu.async_copy(et_h.at[pl.ds(b, CH)], tv[p], sem_e)

        def wait_edges(p):
            pltpu.make_async_copy(src_h.at[pl.ds(0, CH)], sv[p], sem_e).wait()
            pltpu.make_async_copy(dst_h.at[pl.ds(0, CH)], dv[p], sem_e).wait()
            pltpu.make_async_copy(et_h.at[pl.ds(0, CH)], tv[p], sem_e).wait()

        def gq(p):
            for j in range(CH // 16):
                s_ = pl.ds(j * 16, 16)
                tt = tv[p][s_]
                gv[p][s_] = tt * N + sv[p][s_]
                qv[p][s_] = dv[p][s_] * R + tt
                siv[p][s_] = dv[p][s_]

        def issue_gathers(p):
            pltpu.async_copy(y_h.at[gv[p]], rowsv[p], sem_g)
            pltpu.async_copy(w16_h.at[qv[p]], wv[p], sem_g)

        def wait_gathers(p):
            pltpu.make_async_copy(y_h.at[gv[p]], rowsv[p], sem_g).wait()
            pltpu.make_async_copy(w16_h.at[qv[p]], wv[p], sem_g).wait()

        def scale(p):
            for e in range(CH):
                wsp = wv[p][e]
                for j in range(D // 16):
                    s_ = pl.ds(j * 16, 16)
                    rowsv[p][e, s_] = rowsv[p][e, s_] * wsp

        def issue_scatter(p):
            pltpu.async_copy(rowsv[p], acc_sh.at[siv[p]], add=True, sem=semsc[p])

        def wait_scatter(p):
            pltpu.make_async_copy(
                rowsv[p], acc_sh.at[siv[p]], semsc[p]).wait()

        for j in range(CH // 16):
            izero[pl.ds(j * 16, 16)] = jnp.zeros((16,), jnp.int32)
        pltpu.async_copy(zr_v, acc_sh.at[izero], add=True, sem=semsc[1])

        load_edges(0, 0)
        wait_edges(0)
        gq(0)
        issue_gathers(0)
        load_edges(1, 1)

        def body(kk, c):
            wait_edges(1)
            wait_scatter(1)
            gq(1)
            issue_gathers(1)
            wait_gathers(0)
            scale(0)
            issue_scatter(0)
            load_edges(2 * kk + 2, 0)
            wait_edges(0)
            wait_scatter(0)
            gq(0)
            issue_gathers(0)
            wait_gathers(1)
            scale(1)
            issue_scatter(1)

            @pl.when(2 * kk + 3 < NCH)
            def _():
                load_edges(2 * kk + 3, 1)

            return c

        lax.fori_loop(0, (NCH - 1) // 2, body, 0)
        wait_gathers(0)
        scale(0)
        pltpu.sync_copy(rowsv[0], acc_sh.at[siv[0]], add=True)
        wait_scatter(1)
        plsc.subcore_barrier()
        for p in range(NP):
            cidx = sid + p * NS

            @pl.when(cidx < NZC)
            def _():
                pltpu.sync_copy(acc_sh.at[pl.ds(cidx * ZR, ZR)], zr_v)
                pltpu.sync_copy(zr_v, acc_o.at[cid, pl.ds(cidx * ZR, ZR)])

    return k(src, dst, et, yf, w16)


def _tc_weights(comp1, b1f, comp2, b2f, R):
    def body(c1, b1, c2, b2, w1o, w2o):
        w1o[...] = jnp.dot(c1[...], b1[...], preferred_element_type=jnp.float32)
        w2o[...] = jnp.dot(c2[...], b2[...], preferred_element_type=jnp.float32)

    return pl.pallas_call(
        body,
        out_shape=(
            jax.ShapeDtypeStruct((R, b1f.shape[1]), jnp.float32),
            jax.ShapeDtypeStruct((R, b2f.shape[1]), jnp.float32),
        ),
    )(comp1, b1f, comp2, b2f)


def _tc_y1(x2, W1dup, N, D, R, HID):
    BND = 1000
    NT = (N // 2) // BND
    H2 = 2 * HID

    def body(x_ref, w_ref, o_ref):
        o_ref[...] = jnp.dot(x_ref[...], w_ref[...],
                             preferred_element_type=jnp.float32)

    return pl.pallas_call(
        body,
        grid=(NT, R),
        in_specs=[
            pl.BlockSpec((BND, 2 * D), lambda nt, r: (nt, 0)),
            pl.BlockSpec((2 * D, H2), lambda nt, r: (0, r)),
        ],
        out_specs=pl.BlockSpec((BND, H2), lambda nt, r: (r * NT + nt, 0)),
        out_shape=jax.ShapeDtypeStruct((R * N // 2, H2), jnp.float32),
    )(x2, W1dup)


def _tc_h(x, root1, bias1r, acc1, root2p, bias2r, N, D, HID):
    NT = N // BN

    def body(x_ref, r1_ref, b1_ref, a1_ref, r2_ref, b2_ref, h_ref, xr2_ref):
        h = jnp.dot(x_ref[...], r1_ref[...],
                    preferred_element_type=jnp.float32) + b1_ref[...]
        h = h + a1_ref[0] + a1_ref[1]
        h = jnp.maximum(h, 0.0)
        h_ref[...] = h
        xr2_ref[...] = jnp.dot(h, r2_ref[...],
                               preferred_element_type=jnp.float32) + b2_ref[...]

    return pl.pallas_call(
        body,
        grid=(NT,),
        in_specs=[
            pl.BlockSpec((BN, D), lambda nt: (nt, 0)),
            pl.BlockSpec((D, HID), lambda nt: (0, 0)),
            pl.BlockSpec((1, HID), lambda nt: (0, 0)),
            pl.BlockSpec((NC, BN, HID), lambda nt: (0, nt, 0)),
            pl.BlockSpec((HID, 128), lambda nt: (0, 0)),
            pl.BlockSpec((1, 128), lambda nt: (0, 0)),
        ],
        out_specs=(
            pl.BlockSpec((BN, HID), lambda nt: (nt, 0)),
            pl.BlockSpec((BN, 128), lambda nt: (nt, 0)),
        ),
        out_shape=(
            jax.ShapeDtypeStruct((N, HID), jnp.float32),
            jax.ShapeDtypeStruct((N, 128), jnp.float32),
        ),
    )(x, root1, bias1r, acc1, root2p, bias2r)


def _tc_y2(h, W2r, N, HID, R, DO):
    NT = N // BN

    def body(h_ref, w_ref, o_ref):
        o_ref[...] = jnp.dot(h_ref[...], w_ref[0],
                             preferred_element_type=jnp.float32)

    return pl.pallas_call(
        body,
        grid=(NT, R),
        in_specs=[
            pl.BlockSpec((BN, HID), lambda nt, r: (nt, 0)),
            pl.BlockSpec((1, HID, DO), lambda nt, r: (r, 0, 0)),
        ],
        out_specs=pl.BlockSpec((BN, DO), lambda nt, r: (r * NT + nt, 0)),
        out_shape=jax.ShapeDtypeStruct((R * N, DO), jnp.float32),
    )(h, W2r)


def _tc_winv16(cnt2r):
    _, Q8, _ = cnt2r.shape
    NT = 10
    B8 = Q8 // NT

    def body(c_ref, o_ref):
        c = c_ref[0] + c_ref[1]
        w = 1.0 / jnp.maximum(c, 1.0)
        k = lax.broadcasted_iota(jnp.int32, (8, 128), 0)
        cc = lax.broadcasted_iota(jnp.int32, (8, 128), 1)
        sel = (cc // 16 == k).astype(jnp.float32)
        o_ref[...] = jnp.dot(w, sel, preferred_element_type=jnp.float32)

    return pl.pallas_call(
        body,
        grid=(NT,),
        in_specs=[pl.BlockSpec((2, B8, 8), lambda i: (0, i, 0))],
        out_specs=pl.BlockSpec((B8, 128), lambda i: (i, 0)),
        out_shape=jax.ShapeDtypeStruct((Q8, 128), jnp.float32),
    )(cnt2r)


def _tc_logsoftmax(xr2, acc2, N, DO, CLS):
    NT = N // BN

    def body(xr_ref, a_ref, o_ref):
        z = xr_ref[...] + a_ref[0] + a_ref[1]
        col = lax.broadcasted_iota(jnp.int32, z.shape, 1)
        z = jnp.where(col < CLS, z, -1e30)
        m = jnp.max(z, axis=1, keepdims=True)
        e = jnp.exp(z - m)
        s = jnp.sum(e, axis=1, keepdims=True)
        o_ref[...] = z - m - jnp.log(s)

    return pl.pallas_call(
        body,
        grid=(NT,),
        in_specs=[
            pl.BlockSpec((BN, DO), lambda nt: (nt, 0)),
            pl.BlockSpec((NC, BN, DO), lambda nt: (0, nt, 0)),
        ],
        out_specs=pl.BlockSpec((BN, DO), lambda nt: (nt, 0)),
        out_shape=jax.ShapeDtypeStruct((N, DO), jnp.float32),
    )(xr2, acc2)


def kernel(x, edge_index, edge_type, basis1, comp1, root1, bias1,
           basis2, comp2, root2, bias2):
    N, D = x.shape
    HID = root1.shape[1]
    CLS = root2.shape[1]
    R = comp1.shape[0]
    NB = basis1.shape[0]
    E = edge_type.shape[0]
    DO = 128
    RN = R * N

    src = edge_index[0]
    dst = edge_index[1]
    et = edge_type

    b1f = basis1.reshape(NB, D * HID)
    b2p = jnp.pad(basis2, ((0, 0), (0, 0), (0, DO - CLS)))
    b2f = b2p.reshape(NB, HID * DO)
    root2p = jnp.pad(root2, ((0, 0), (0, DO - CLS)))
    bias2p = jnp.pad(bias2, (0, DO - CLS)).reshape(1, DO)
    bias1r = bias1.reshape(1, HID)

    W1f, W2f = _tc_weights(comp1, b1f, comp2, b2f, R)
    W1r = W1f.reshape(R, D, HID)
    W1dup = jnp.einsum('ab,rdj->adrbj', jnp.eye(2, dtype=x.dtype),
                       W1r).reshape(2 * D, R * 2 * HID)
    W2r = W2f.reshape(R, HID, DO)

    cnt2 = _sc_count(dst, et, N, R, E)
    winv16 = _tc_winv16(cnt2.reshape(NC, RN // 8, 8)).reshape(RN, 16)

    x2 = x.reshape(N // 2, 2 * D)
    y1 = _tc_y1(x2, W1dup, N, D, R, HID)
    acc1 = _sc_pass(src, dst, et, y1.reshape(RN, HID), winv16, N, R, E, HID)

    h, xr2 = _tc_h(x, root1, bias1r, acc1, root2p, bias2p, N, D, HID)
    y2 = _tc_y2(h, W2r, N, HID, R, DO)
    acc2 = _sc_pass(src, dst, et, y2, winv16, N, R, E, DO)

    out = _tc_logsoftmax(xr2, acc2, N, DO, CLS)
    return out[:, :CLS]


# speedup vs baseline: 1.1682x; 1.1682x over previous
import functools

import jax
import jax.numpy as jnp
from jax import lax
from jax.experimental import pallas as pl
from jax.experimental.pallas import tpu as pltpu
from jax.experimental.pallas import tpu_sc as plsc

NC = 2
NS = 16
NW = NC * NS
CH = 80
BN = 1000

_mesh = plsc.VectorSubcoreMesh(core_axis_name="c", subcore_axis_name="s")


def _sc_count(dst, et, N, R, E):
    RN = R * N
    EPW = E // NW
    NCH = EPW // CH
    ZB = RN // NS

    @functools.partial(
        pl.kernel,
        out_type=jax.ShapeDtypeStruct((NC * RN,), jnp.float32),
        mesh=_mesh,
        compiler_params=pltpu.CompilerParams(use_tc_tiling_on_sc=False),
        scratch_types=[
            pltpu.VMEM_SHARED((RN,), jnp.float32),
            pltpu.VMEM((CH,), jnp.int32),
            pltpu.VMEM((CH,), jnp.int32),
            pltpu.VMEM((CH,), jnp.int32),
            pltpu.VMEM((CH,), jnp.int32),
            pltpu.VMEM((CH,), jnp.int32),
            pltpu.VMEM((CH,), jnp.int32),
            pltpu.VMEM((CH,), jnp.float32),
            pltpu.VMEM((CH,), jnp.float32),
            pltpu.VMEM((ZB,), jnp.float32),
            pltpu.SemaphoreType.DMA,
            pltpu.SemaphoreType.DMA,
            pltpu.SemaphoreType.DMA,
        ],
    )
    def k(dst_h, et_h, cnt_o, cnt_sh,
          d_v0, t_v0, i_v0, d_v1, t_v1, i_v1, ones_v, zc_v, z_v,
          sem_e, sem_c0, sem_c1):
        cid = lax.axis_index("c")
        sid = lax.axis_index("s")
        wid = sid * NC + cid
        dv = (d_v0, d_v1)
        tv = (t_v0, t_v1)
        iv = (i_v0, i_v1)
        semc = (sem_c0, sem_c1)

        def zlp(j, c):
            z_v[pl.ds(j * 16, 16)] = jnp.zeros((16,), jnp.float32)
            return c

        lax.fori_loop(0, ZB // 16, zlp, 0)
        pltpu.sync_copy(z_v, cnt_sh.at[pl.ds(sid * ZB, ZB)])
        for j in range(CH // 16):
            s_ = pl.ds(j * 16, 16)
            ones_v[s_] = jnp.ones((16,), jnp.float32)
            zc_v[s_] = jnp.zeros((16,), jnp.float32)
            i_v0[s_] = jnp.zeros((16,), jnp.int32)
            i_v1[s_] = jnp.zeros((16,), jnp.int32)
        pltpu.async_copy(zc_v, cnt_sh.at[i_v0], add=True, sem=sem_c0)
        pltpu.async_copy(zc_v, cnt_sh.at[i_v1], add=True, sem=sem_c1)
        plsc.subcore_barrier()

        def load_edges(i, p):
            b = pl.multiple_of(wid * EPW + i * CH, 8)
            pltpu.async_copy(dst_h.at[pl.ds(b, CH)], dv[p], sem_e)
            pltpu.async_copy(et_h.at[pl.ds(b, CH)], tv[p], sem_e)

        def wait_edges(p):
            pltpu.make_async_copy(dst_h.at[pl.ds(0, CH)], dv[p], sem_e).wait()
            pltpu.make_async_copy(et_h.at[pl.ds(0, CH)], tv[p], sem_e).wait()

        def step(i, p, po):
            wait_edges(p)
            pltpu.make_async_copy(ones_v, cnt_sh.at[iv[p]], semc[p]).wait()
            for j in range(CH // 16):
                s_ = pl.ds(j * 16, 16)
                iv[p][s_] = dv[p][s_] * R + tv[p][s_]

            @pl.when(i + 1 < NCH)
            def _():
                load_edges(i + 1, po)

            pltpu.async_copy(ones_v, cnt_sh.at[iv[p]], add=True, sem=semc[p])

        load_edges(0, 0)

        def body(kk, c):
            step(2 * kk, 0, 1)
            step(2 * kk + 1, 1, 0)
            return c

        lax.fori_loop(0, (NCH - 1) // 2, body, 0)
        step(NCH - 1, 0, 1)
        pltpu.make_async_copy(ones_v, cnt_sh.at[i_v0], sem_c0).wait()
        pltpu.make_async_copy(ones_v, cnt_sh.at[i_v1], sem_c1).wait()
        plsc.subcore_barrier()
        pltpu.sync_copy(cnt_sh.at[pl.ds(sid * ZB, ZB)], z_v)
        pltpu.sync_copy(z_v, cnt_o.at[pl.ds(cid * RN + sid * ZB, ZB)])

    return k(dst, et)


def _sc_pass(src, dst, et, yf, w16, N, R, E, D):
    EPW = E // NW
    NCH = EPW // CH
    assert NCH % 2 == 1
    ZR = 80
    NZC = N // ZR
    NP = (NZC + NS - 1) // NS

    @functools.partial(
        pl.kernel,
        out_type=jax.ShapeDtypeStruct((NC, N, D), jnp.float32),
        mesh=_mesh,
        compiler_params=pltpu.CompilerParams(use_tc_tiling_on_sc=False),
        scratch_types=[
            pltpu.VMEM_SHARED((N, D), jnp.float32),
            pltpu.VMEM((CH,), jnp.int32),
            pltpu.VMEM((CH,), jnp.int32),
            pltpu.VMEM((CH,), jnp.int32),
            pltpu.VMEM((CH,), jnp.int32),
            pltpu.VMEM((CH,), jnp.int32),
            pltpu.VMEM((CH,), jnp.int32),
            pltpu.VMEM((CH,), jnp.int32),
            pltpu.VMEM((CH,), jnp.int32),
            pltpu.VMEM((CH,), jnp.int32),
            pltpu.VMEM((CH,), jnp.int32),
            pltpu.VMEM((CH, 16), jnp.float32),
            pltpu.VMEM((CH, 16), jnp.float32),
            pltpu.VMEM((CH, D), jnp.float32),
            pltpu.VMEM((CH, D), jnp.float32),
            pltpu.VMEM((ZR, D), jnp.float32),
            pltpu.VMEM((CH,), jnp.int32),
            pltpu.VMEM((CH,), jnp.int32),
            pltpu.VMEM((CH,), jnp.int32),
            pltpu.SemaphoreType.DMA,
            pltpu.SemaphoreType.DMA,
            pltpu.SemaphoreType.DMA,
            pltpu.SemaphoreType.DMA,
        ],
    )
    def k(src_h, dst_h, et_h, y_h, w16_h, acc_o, acc_sh,
          s_v0, d_v0, t_v0, g_v0, q_v0,
          s_v1, d_v1, t_v1, g_v1, q_v1,
          w_v0, w_v1, rows_v0, rows_v1, zr_v, si_v0, si_v1, izero,
          sem_e, sem_g, sem_sc0, sem_sc1):
        cid = lax.axis_index("c")
        sid = lax.axis_index("s")
        wid = sid * NC + cid
        sv = (s_v0, s_v1)
        dv = (d_v0, d_v1)
        tv = (t_v0, t_v1)
        gv = (g_v0, g_v1)
        qv = (q_v0, q_v1)
        wv = (w_v0, w_v1)
        rowsv = (rows_v0, rows_v1)
        siv = (si_v0, si_v1)
        semsc = (sem_sc0, sem_sc1)

        def zlp(i, c):
            for j in range(D // 16):
                zr_v[i, pl.ds(j * 16, 16)] = jnp.zeros((16,), jnp.float32)
            return c

        lax.fori_loop(0, ZR, zlp, 0)
        for p in range(NP):
            cidx = sid + p * NS

            @pl.when(cidx < NZC)
            def _():
                pltpu.sync_copy(zr_v, acc_sh.at[pl.ds(cidx * ZR, ZR)])

        plsc.subcore_barrier()

        def load_edges(i, p):
            b = pl.multiple_of(wid * EPW + i * CH, 8)
            pltpu.async_copy(src_h.at[pl.ds(b, CH)], sv[p], sem_e)
            pltpu.async_copy(dst_h.at[pl.ds(b, CH)], dv[p], sem_e)
            pltpu.async_copy(et_h.at[pl.ds(b, CH)], tv[p], sem_e)

        def wait_edges(p):
            pltpu.make_async_copy(src_h.at[pl.ds(0, CH)], sv[p], sem_e).wait()
            pltpu.make_async_copy(dst_h.at[pl.ds(0, CH)], dv[p], sem_e).wait()
            pltpu.make_async_copy(et_h.at[pl.ds(0, CH)], tv[p], sem_e).wait()

        def gq(p):
            for j in range(CH // 16):
                s_ = pl.ds(j * 16, 16)
                tt = tv[p][s_]
                gv[p][s_] = tt * N + sv[p][s_]
                qv[p][s_] = dv[p][s_] * R + tt
                siv[p][s_] = dv[p][s_]

        def issue_gathers(p):
            pltpu.async_copy(y_h.at[gv[p]], rowsv[p], sem_g)
            pltpu.async_copy(w16_h.at[qv[p]], wv[p], sem_g)

        def wait_gathers(p):
            pltpu.make_async_copy(y_h.at[gv[p]], rowsv[p], sem_g).wait()
            pltpu.make_async_copy(w16_h.at[qv[p]], wv[p], sem_g).wait()

        def scale(p):
            for e in range(CH):
                wsp = wv[p][e]
                for j in range(D // 16):
                    s_ = pl.ds(j * 16, 16)
                    rowsv[p][e, s_] = rowsv[p][e, s_] * wsp

        def issue_scatter(p):
            pltpu.async_copy(rowsv[p], acc_sh.at[siv[p]], add=True, sem=semsc[p])

        def wait_scatter(p):
            pltpu.make_async_copy(
                rowsv[p], acc_sh.at[siv[p]], semsc[p]).wait()

        for j in range(CH // 16):
            izero[pl.ds(j * 16, 16)] = jnp.zeros((16,), jnp.int32)
        pltpu.async_copy(zr_v, acc_sh.at[izero], add=True, sem=semsc[1])

        load_edges(0, 0)
        wait_edges(0)
        gq(0)
        issue_gathers(0)
        load_edges(1, 1)

        def body(kk, c):
            wait_edges(1)
            wait_scatter(1)
            gq(1)
            issue_gathers(1)
            wait_gathers(0)
            scale(0)
            issue_scatter(0)
            load_edges(2 * kk + 2, 0)
            wait_edges(0)
            wait_scatter(0)
            gq(0)
            issue_gathers(0)
            wait_gathers(1)
            scale(1)
            issue_scatter(1)

            @pl.when(2 * kk + 3 < NCH)
            def _():
                load_edges(2 * kk + 3, 1)

            return c

        lax.fori_loop(0, (NCH - 1) // 2, body, 0)
        wait_gathers(0)
        scale(0)
        pltpu.sync_copy(rowsv[0], acc_sh.at[siv[0]], add=True)
        wait_scatter(1)
        plsc.subcore_barrier()
        for p in range(NP):
            cidx = sid + p * NS

            @pl.when(cidx < NZC)
            def _():
                pltpu.sync_copy(acc_sh.at[pl.ds(cidx * ZR, ZR)], zr_v)
                pltpu.sync_copy(zr_v, acc_o.at[cid, pl.ds(cidx * ZR, ZR)])

    return k(src, dst, et, yf, w16)


def _tc_weights(comp1, b1f, comp2, b2f, R):
    def body(c1, b1, c2, b2, w1o, w2o):
        w1o[...] = jnp.dot(c1[...], b1[...], preferred_element_type=jnp.float32)
        w2o[...] = jnp.dot(c2[...], b2[...], preferred_element_type=jnp.float32)

    return pl.pallas_call(
        body,
        out_shape=(
            jax.ShapeDtypeStruct((R, b1f.shape[1]), jnp.float32),
            jax.ShapeDtypeStruct((R, b2f.shape[1]), jnp.float32),
        ),
    )(comp1, b1f, comp2, b2f)


def _tc_y1(x2, W1dup, N, D, R, HID):
    H2 = 2 * HID

    def body(x_ref, w_ref, o_ref):
        o_ref[...] = jnp.dot(x_ref[...], w_ref[...],
                             preferred_element_type=jnp.float32)

    return pl.pallas_call(
        body,
        grid=(R,),
        in_specs=[
            pl.BlockSpec((N // 2, 2 * D), lambda r: (0, 0)),
            pl.BlockSpec((2 * D, H2), lambda r: (0, r)),
        ],
        out_specs=pl.BlockSpec((N // 2, H2), lambda r: (r, 0)),
        out_shape=jax.ShapeDtypeStruct((R * N // 2, H2), jnp.float32),
    )(x2, W1dup)


def _tc_h(x, root1, bias1r, acc1, root2p, bias2r, N, D, HID):
    NT = N // BN

    def body(x_ref, r1_ref, b1_ref, a1_ref, r2_ref, b2_ref, h_ref, xr2_ref):
        h = jnp.dot(x_ref[...], r1_ref[...],
                    preferred_element_type=jnp.float32) + b1_ref[...]
        h = h + a1_ref[0] + a1_ref[1]
        h = jnp.maximum(h, 0.0)
        h_ref[...] = h
        xr2_ref[...] = jnp.dot(h, r2_ref[...],
                               preferred_element_type=jnp.float32) + b2_ref[...]

    return pl.pallas_call(
        body,
        grid=(NT,),
        in_specs=[
            pl.BlockSpec((BN, D), lambda nt: (nt, 0)),
            pl.BlockSpec((D, HID), lambda nt: (0, 0)),
            pl.BlockSpec((1, HID), lambda nt: (0, 0)),
            pl.BlockSpec((NC, BN, HID), lambda nt: (0, nt, 0)),
            pl.BlockSpec((HID, 128), lambda nt: (0, 0)),
            pl.BlockSpec((1, 128), lambda nt: (0, 0)),
        ],
        out_specs=(
            pl.BlockSpec((BN, HID), lambda nt: (nt, 0)),
            pl.BlockSpec((BN, 128), lambda nt: (nt, 0)),
        ),
        out_shape=(
            jax.ShapeDtypeStruct((N, HID), jnp.float32),
            jax.ShapeDtypeStruct((N, 128), jnp.float32),
        ),
    )(x, root1, bias1r, acc1, root2p, bias2r)


def _tc_y2(h, W2r, N, HID, R, DO):
    def body(h_ref, w_ref, o_ref):
        o_ref[...] = jnp.dot(h_ref[...], w_ref[0],
                             preferred_element_type=jnp.float32)

    return pl.pallas_call(
        body,
        grid=(R,),
        in_specs=[
            pl.BlockSpec((N, HID), lambda r: (0, 0)),
            pl.BlockSpec((1, HID, DO), lambda r: (r, 0, 0)),
        ],
        out_specs=pl.BlockSpec((N, DO), lambda r: (r, 0)),
        out_shape=jax.ShapeDtypeStruct((R * N, DO), jnp.float32),
    )(h, W2r)


def _tc_winv16(cnt2r):
    _, Q8, _ = cnt2r.shape
    NT = 10
    B8 = Q8 // NT

    def body(c_ref, o_ref):
        c = c_ref[0] + c_ref[1]
        w = 1.0 / jnp.maximum(c, 1.0)
        k = lax.broadcasted_iota(jnp.int32, (8, 128), 0)
        cc = lax.broadcasted_iota(jnp.int32, (8, 128), 1)
        sel = (cc // 16 == k).astype(jnp.float32)
        o_ref[...] = jnp.dot(w, sel, preferred_element_type=jnp.float32)

    return pl.pallas_call(
        body,
        grid=(NT,),
        in_specs=[pl.BlockSpec((2, B8, 8), lambda i: (0, i, 0))],
        out_specs=pl.BlockSpec((B8, 128), lambda i: (i, 0)),
        out_shape=jax.ShapeDtypeStruct((Q8, 128), jnp.float32),
    )(cnt2r)


def _tc_logsoftmax(xr2, acc2, N, DO, CLS):
    NT = N // BN

    def body(xr_ref, a_ref, o_ref):
        z = xr_ref[...] + a_ref[0] + a_ref[1]
        col = lax.broadcasted_iota(jnp.int32, z.shape, 1)
        z = jnp.where(col < CLS, z, -1e30)
        m = jnp.max(z, axis=1, keepdims=True)
        e = jnp.exp(z - m)
        s = jnp.sum(e, axis=1, keepdims=True)
        o_ref[...] = z - m - jnp.log(s)

    return pl.pallas_call(
        body,
        grid=(NT,),
        in_specs=[
            pl.BlockSpec((BN, DO), lambda nt: (nt, 0)),
            pl.BlockSpec((NC, BN, DO), lambda nt: (0, nt, 0)),
        ],
        out_specs=pl.BlockSpec((BN, DO), lambda nt: (nt, 0)),
        out_shape=jax.ShapeDtypeStruct((N, DO), jnp.float32),
    )(xr2, acc2)


def kernel(x, edge_index, edge_type, basis1, comp1, root1, bias1,
           basis2, comp2, root2, bias2):
    N, D = x.shape
    HID = root1.shape[1]
    CLS = root2.shape[1]
    R = comp1.shape[0]
    NB = basis1.shape[0]
    E = edge_type.shape[0]
    DO = 128
    RN = R * N

    src = edge_index[0]
    dst = edge_index[1]
    et = edge_type

    b1f = basis1.reshape(NB, D * HID)
    b2p = jnp.pad(basis2, ((0, 0), (0, 0), (0, DO - CLS)))
    b2f = b2p.reshape(NB, HID * DO)
    root2p = jnp.pad(root2, ((0, 0), (0, DO - CLS)))
    bias2p = jnp.pad(bias2, (0, DO - CLS)).reshape(1, DO)
    bias1r = bias1.reshape(1, HID)

    W1f, W2f = _tc_weights(comp1, b1f, comp2, b2f, R)
    W1r = W1f.reshape(R, D, HID)
    W1dup = jnp.einsum('ab,rdj->adrbj', jnp.eye(2, dtype=x.dtype),
                       W1r).reshape(2 * D, R * 2 * HID)
    W2r = W2f.reshape(R, HID, DO)

    cnt2 = _sc_count(dst, et, N, R, E)
    winv16 = _tc_winv16(cnt2.reshape(NC, RN // 8, 8)).reshape(RN, 16)

    x2 = x.reshape(N // 2, 2 * D)
    y1 = _tc_y1(x2, W1dup, N, D, R, HID)
    acc1 = _sc_pass(src, dst, et, y1.reshape(RN, HID), winv16, N, R, E, HID)

    h, xr2 = _tc_h(x, root1, bias1r, acc1, root2p, bias2p, N, D, HID)
    y2 = _tc_y2(h, W2r, N, HID, R, DO)
    acc2 = _sc_pass(src, dst, et, y2, winv16, N, R, E, DO)

    out = _tc_logsoftmax(xr2, acc2, N, DO, CLS)
    return out[:, :CLS]


# speedup vs baseline: 1.1909x; 1.0194x over previous
import functools

import jax
import jax.numpy as jnp
from jax import lax
from jax.experimental import pallas as pl
from jax.experimental.pallas import tpu as pltpu
from jax.experimental.pallas import tpu_sc as plsc

NC = 2
NS = 16
NW = NC * NS
CH = 80
BN = 1000

_mesh = plsc.VectorSubcoreMesh(core_axis_name="c", subcore_axis_name="s")


def _sc_count(ei, et, N, R, E):
    RN = R * N
    EPW = E // NW
    NCH = EPW // CH
    ZB = RN // NS

    @functools.partial(
        pl.kernel,
        out_type=jax.ShapeDtypeStruct((NC * RN,), jnp.float32),
        mesh=_mesh,
        compiler_params=pltpu.CompilerParams(use_tc_tiling_on_sc=False),
        scratch_types=[
            pltpu.VMEM_SHARED((RN,), jnp.float32),
            pltpu.VMEM((CH,), jnp.int32),
            pltpu.VMEM((CH,), jnp.int32),
            pltpu.VMEM((CH,), jnp.int32),
            pltpu.VMEM((CH,), jnp.int32),
            pltpu.VMEM((CH,), jnp.int32),
            pltpu.VMEM((CH,), jnp.int32),
            pltpu.VMEM((CH,), jnp.float32),
            pltpu.VMEM((CH,), jnp.float32),
            pltpu.VMEM((ZB,), jnp.float32),
            pltpu.SemaphoreType.DMA,
            pltpu.SemaphoreType.DMA,
            pltpu.SemaphoreType.DMA,
        ],
    )
    def k(ei_h, et_h, cnt_o, cnt_sh,
          d_v0, t_v0, i_v0, d_v1, t_v1, i_v1, ones_v, zc_v, z_v,
          sem_e, sem_c0, sem_c1):
        cid = lax.axis_index("c")
        sid = lax.axis_index("s")
        wid = sid * NC + cid
        dv = (d_v0, d_v1)
        tv = (t_v0, t_v1)
        iv = (i_v0, i_v1)
        semc = (sem_c0, sem_c1)

        def zlp(j, c):
            z_v[pl.ds(j * 16, 16)] = jnp.zeros((16,), jnp.float32)
            return c

        lax.fori_loop(0, ZB // 16, zlp, 0)
        pltpu.sync_copy(z_v, cnt_sh.at[pl.ds(sid * ZB, ZB)])
        for j in range(CH // 16):
            s_ = pl.ds(j * 16, 16)
            ones_v[s_] = jnp.ones((16,), jnp.float32)
            zc_v[s_] = jnp.zeros((16,), jnp.float32)
            i_v0[s_] = jnp.zeros((16,), jnp.int32)
            i_v1[s_] = jnp.zeros((16,), jnp.int32)
        pltpu.async_copy(zc_v, cnt_sh.at[i_v0], add=True, sem=sem_c0)
        pltpu.async_copy(zc_v, cnt_sh.at[i_v1], add=True, sem=sem_c1)
        plsc.subcore_barrier()

        def load_edges(i, p):
            b = pl.multiple_of(wid * EPW + i * CH, 8)
            pltpu.async_copy(ei_h.at[1, pl.ds(b, CH)], dv[p], sem_e)
            pltpu.async_copy(et_h.at[pl.ds(b, CH)], tv[p], sem_e)

        def wait_edges(p):
            pltpu.make_async_copy(
                ei_h.at[1, pl.ds(0, CH)], dv[p], sem_e).wait()
            pltpu.make_async_copy(et_h.at[pl.ds(0, CH)], tv[p], sem_e).wait()

        def step(i, p, po):
            wait_edges(p)
            pltpu.make_async_copy(ones_v, cnt_sh.at[iv[p]], semc[p]).wait()
            for j in range(CH // 16):
                s_ = pl.ds(j * 16, 16)
                iv[p][s_] = dv[p][s_] * R + tv[p][s_]

            @pl.when(i + 1 < NCH)
            def _():
                load_edges(i + 1, po)

            pltpu.async_copy(ones_v, cnt_sh.at[iv[p]], add=True, sem=semc[p])

        load_edges(0, 0)

        def body(kk, c):
            step(2 * kk, 0, 1)
            step(2 * kk + 1, 1, 0)
            return c

        lax.fori_loop(0, (NCH - 1) // 2, body, 0)
        step(NCH - 1, 0, 1)
        pltpu.make_async_copy(ones_v, cnt_sh.at[i_v0], sem_c0).wait()
        pltpu.make_async_copy(ones_v, cnt_sh.at[i_v1], sem_c1).wait()
        plsc.subcore_barrier()
        pltpu.sync_copy(cnt_sh.at[pl.ds(sid * ZB, ZB)], z_v)
        pltpu.sync_copy(z_v, cnt_o.at[pl.ds(cid * RN + sid * ZB, ZB)])

    return k(ei, et)


def _sc_pass(ei, et, yf, w16, N, R, E, D):
    EPW = E // NW
    NCH = EPW // CH
    assert NCH % 2 == 1
    ZR = 80
    NZC = N // ZR
    NP = (NZC + NS - 1) // NS

    @functools.partial(
        pl.kernel,
        out_type=jax.ShapeDtypeStruct((NC, N, D), jnp.float32),
        mesh=_mesh,
        compiler_params=pltpu.CompilerParams(use_tc_tiling_on_sc=False),
        scratch_types=[
            pltpu.VMEM_SHARED((N, D), jnp.float32),
            pltpu.VMEM((CH,), jnp.int32),
            pltpu.VMEM((CH,), jnp.int32),
            pltpu.VMEM((CH,), jnp.int32),
            pltpu.VMEM((CH,), jnp.int32),
            pltpu.VMEM((CH,), jnp.int32),
            pltpu.VMEM((CH,), jnp.int32),
            pltpu.VMEM((CH,), jnp.int32),
            pltpu.VMEM((CH,), jnp.int32),
            pltpu.VMEM((CH,), jnp.int32),
            pltpu.VMEM((CH,), jnp.int32),
            pltpu.VMEM((CH, 16), jnp.float32),
            pltpu.VMEM((CH, 16), jnp.float32),
            pltpu.VMEM((CH, D), jnp.float32),
            pltpu.VMEM((CH, D), jnp.float32),
            pltpu.VMEM((ZR, D), jnp.float32),
            pltpu.VMEM((CH,), jnp.int32),
            pltpu.VMEM((CH,), jnp.int32),
            pltpu.VMEM((CH,), jnp.int32),
            pltpu.SemaphoreType.DMA,
            pltpu.SemaphoreType.DMA,
            pltpu.SemaphoreType.DMA,
            pltpu.SemaphoreType.DMA,
        ],
    )
    def k(ei_h, et_h, y_h, w16_h, acc_o, acc_sh,
          s_v0, d_v0, t_v0, g_v0, q_v0,
          s_v1, d_v1, t_v1, g_v1, q_v1,
          w_v0, w_v1, rows_v0, rows_v1, zr_v, si_v0, si_v1, izero,
          sem_e, sem_g, sem_sc0, sem_sc1):
        cid = lax.axis_index("c")
        sid = lax.axis_index("s")
        wid = sid * NC + cid
        sv = (s_v0, s_v1)
        dv = (d_v0, d_v1)
        tv = (t_v0, t_v1)
        gv = (g_v0, g_v1)
        qv = (q_v0, q_v1)
        wv = (w_v0, w_v1)
        rowsv = (rows_v0, rows_v1)
        siv = (si_v0, si_v1)
        semsc = (sem_sc0, sem_sc1)

        def zlp(i, c):
            for j in range(D // 16):
                zr_v[i, pl.ds(j * 16, 16)] = jnp.zeros((16,), jnp.float32)
            return c

        lax.fori_loop(0, ZR, zlp, 0)
        for p in range(NP):
            cidx = sid + p * NS

            @pl.when(cidx < NZC)
            def _():
                pltpu.sync_copy(zr_v, acc_sh.at[pl.ds(cidx * ZR, ZR)])

        plsc.subcore_barrier()

        def load_edges(i, p):
            b = pl.multiple_of(wid * EPW + i * CH, 8)
            pltpu.async_copy(ei_h.at[0, pl.ds(b, CH)], sv[p], sem_e)
            pltpu.async_copy(ei_h.at[1, pl.ds(b, CH)], dv[p], sem_e)
            pltpu.async_copy(et_h.at[pl.ds(b, CH)], tv[p], sem_e)

        def wait_edges(p):
            pltpu.make_async_copy(
                ei_h.at[0, pl.ds(0, CH)], sv[p], sem_e).wait()
            pltpu.make_async_copy(
                ei_h.at[1, pl.ds(0, CH)], dv[p], sem_e).wait()
            pltpu.make_async_copy(et_h.at[pl.ds(0, CH)], tv[p], sem_e).wait()

        def gq(p):
            for j in range(CH // 16):
                s_ = pl.ds(j * 16, 16)
                tt = tv[p][s_]
                gv[p][s_] = tt * N + sv[p][s_]
                qv[p][s_] = dv[p][s_] * R + tt
                siv[p][s_] = dv[p][s_]

        def issue_gathers(p):
            pltpu.async_copy(y_h.at[gv[p]], rowsv[p], sem_g)
            pltpu.async_copy(w16_h.at[qv[p]], wv[p], sem_g)

        def wait_gathers(p):
            pltpu.make_async_copy(y_h.at[gv[p]], rowsv[p], sem_g).wait()
            pltpu.make_async_copy(w16_h.at[qv[p]], wv[p], sem_g).wait()

        def scale(p):
            for e in range(CH):
                wsp = wv[p][e]
                for j in range(D // 16):
                    s_ = pl.ds(j * 16, 16)
                    rowsv[p][e, s_] = rowsv[p][e, s_] * wsp

        def issue_scatter(p):
            pltpu.async_copy(rowsv[p], acc_sh.at[siv[p]], add=True, sem=semsc[p])

        def wait_scatter(p):
            pltpu.make_async_copy(
                rowsv[p], acc_sh.at[siv[p]], semsc[p]).wait()

        for j in range(CH // 16):
            izero[pl.ds(j * 16, 16)] = jnp.zeros((16,), jnp.int32)
        pltpu.async_copy(zr_v, acc_sh.at[izero], add=True, sem=semsc[1])

        load_edges(0, 0)
        wait_edges(0)
        gq(0)
        issue_gathers(0)
        load_edges(1, 1)

        def body(kk, c):
            wait_edges(1)
            wait_scatter(1)
            gq(1)
            issue_gathers(1)
            wait_gathers(0)
            scale(0)
            issue_scatter(0)
            load_edges(2 * kk + 2, 0)
            wait_edges(0)
            wait_scatter(0)
            gq(0)
            issue_gathers(0)
            wait_gathers(1)
            scale(1)
            issue_scatter(1)

            @pl.when(2 * kk + 3 < NCH)
            def _():
                load_edges(2 * kk + 3, 1)

            return c

        lax.fori_loop(0, (NCH - 1) // 2, body, 0)
        wait_gathers(0)
        scale(0)
        pltpu.sync_copy(rowsv[0], acc_sh.at[siv[0]], add=True)
        wait_scatter(1)
        plsc.subcore_barrier()
        for p in range(NP):
            cidx = sid + p * NS

            @pl.when(cidx < NZC)
            def _():
                pltpu.sync_copy(acc_sh.at[pl.ds(cidx * ZR, ZR)], zr_v)
                pltpu.sync_copy(zr_v, acc_o.at[cid, pl.ds(cidx * ZR, ZR)])

    return k(ei, et, yf, w16)


def _tc_weights(comp1, b1f, comp2, b2f, R):
    def body(c1, b1, c2, b2, w1o, w2o):
        w1o[...] = jnp.dot(c1[...], b1[...], preferred_element_type=jnp.float32)
        w2o[...] = jnp.dot(c2[...], b2[...], preferred_element_type=jnp.float32)

    return pl.pallas_call(
        body,
        out_shape=(
            jax.ShapeDtypeStruct((R, b1f.shape[1]), jnp.float32),
            jax.ShapeDtypeStruct((R, b2f.shape[1]), jnp.float32),
        ),
    )(comp1, b1f, comp2, b2f)


def _tc_y1(x2, W1dup, N, D, R, HID):
    H2 = 2 * HID

    def body(x_ref, w_ref, o_ref):
        o_ref[...] = jnp.dot(x_ref[...], w_ref[...],
                             preferred_element_type=jnp.float32)

    return pl.pallas_call(
        body,
        grid=(R,),
        in_specs=[
            pl.BlockSpec((N // 2, 2 * D), lambda r: (0, 0)),
            pl.BlockSpec((2 * D, H2), lambda r: (0, r)),
        ],
        out_specs=pl.BlockSpec((N // 2, H2), lambda r: (r, 0)),
        out_shape=jax.ShapeDtypeStruct((R * N // 2, H2), jnp.float32),
    )(x2, W1dup)


def _tc_h(x, root1, bias1r, acc1, root2p, bias2r, N, D, HID):
    NT = N // BN

    def body(x_ref, r1_ref, b1_ref, a1_ref, r2_ref, b2_ref, h_ref, xr2_ref):
        h = jnp.dot(x_ref[...], r1_ref[...],
                    preferred_element_type=jnp.float32) + b1_ref[...]
        h = h + a1_ref[0] + a1_ref[1]
        h = jnp.maximum(h, 0.0)
        h_ref[...] = h
        xr2_ref[...] = jnp.dot(h, r2_ref[...],
                               preferred_element_type=jnp.float32) + b2_ref[...]

    return pl.pallas_call(
        body,
        grid=(NT,),
        in_specs=[
            pl.BlockSpec((BN, D), lambda nt: (nt, 0)),
            pl.BlockSpec((D, HID), lambda nt: (0, 0)),
            pl.BlockSpec((1, HID), lambda nt: (0, 0)),
            pl.BlockSpec((NC, BN, HID), lambda nt: (0, nt, 0)),
            pl.BlockSpec((HID, 128), lambda nt: (0, 0)),
            pl.BlockSpec((1, 128), lambda nt: (0, 0)),
        ],
        out_specs=(
            pl.BlockSpec((BN, HID), lambda nt: (nt, 0)),
            pl.BlockSpec((BN, 128), lambda nt: (nt, 0)),
        ),
        out_shape=(
            jax.ShapeDtypeStruct((N, HID), jnp.float32),
            jax.ShapeDtypeStruct((N, 128), jnp.float32),
        ),
    )(x, root1, bias1r, acc1, root2p, bias2r)


def _tc_y2(h, W2r, N, HID, R, DO):
    def body(h_ref, w_ref, o_ref):
        o_ref[...] = jnp.dot(h_ref[...], w_ref[0],
                             preferred_element_type=jnp.float32)

    return pl.pallas_call(
        body,
        grid=(R,),
        in_specs=[
            pl.BlockSpec((N, HID), lambda r: (0, 0)),
            pl.BlockSpec((1, HID, DO), lambda r: (r, 0, 0)),
        ],
        out_specs=pl.BlockSpec((N, DO), lambda r: (r, 0)),
        out_shape=jax.ShapeDtypeStruct((R * N, DO), jnp.float32),
    )(h, W2r)


def _tc_winv16(cnt2r):
    _, Q8, _ = cnt2r.shape
    NT = 10
    B8 = Q8 // NT

    def body(c_ref, o_ref):
        c = c_ref[0] + c_ref[1]
        w = 1.0 / jnp.maximum(c, 1.0)
        k = lax.broadcasted_iota(jnp.int32, (8, 128), 0)
        cc = lax.broadcasted_iota(jnp.int32, (8, 128), 1)
        sel = (cc // 16 == k).astype(jnp.float32)
        o_ref[...] = jnp.dot(w, sel, preferred_element_type=jnp.float32)

    return pl.pallas_call(
        body,
        grid=(NT,),
        in_specs=[pl.BlockSpec((2, B8, 8), lambda i: (0, i, 0))],
        out_specs=pl.BlockSpec((B8, 128), lambda i: (i, 0)),
        out_shape=jax.ShapeDtypeStruct((Q8, 128), jnp.float32),
    )(cnt2r)


def _tc_logsoftmax(xr2, acc2, N, DO, CLS):
    NT = N // BN

    def body(xr_ref, a_ref, o_ref):
        z = xr_ref[...] + a_ref[0] + a_ref[1]
        col = lax.broadcasted_iota(jnp.int32, z.shape, 1)
        z = jnp.where(col < CLS, z, -1e30)
        m = jnp.max(z, axis=1, keepdims=True)
        e = jnp.exp(z - m)
        s = jnp.sum(e, axis=1, keepdims=True)
        o_ref[...] = z - m - jnp.log(s)

    return pl.pallas_call(
        body,
        grid=(NT,),
        in_specs=[
            pl.BlockSpec((BN, DO), lambda nt: (nt, 0)),
            pl.BlockSpec((NC, BN, DO), lambda nt: (0, nt, 0)),
        ],
        out_specs=pl.BlockSpec((BN, DO), lambda nt: (nt, 0)),
        out_shape=jax.ShapeDtypeStruct((N, DO), jnp.float32),
    )(xr2, acc2)


def kernel(x, edge_index, edge_type, basis1, comp1, root1, bias1,
           basis2, comp2, root2, bias2):
    N, D = x.shape
    HID = root1.shape[1]
    CLS = root2.shape[1]
    R = comp1.shape[0]
    NB = basis1.shape[0]
    E = edge_type.shape[0]
    DO = 128
    RN = R * N

    et = edge_type

    b1f = basis1.reshape(NB, D * HID)
    b2p = jnp.pad(basis2, ((0, 0), (0, 0), (0, DO - CLS)))
    b2f = b2p.reshape(NB, HID * DO)
    root2p = jnp.pad(root2, ((0, 0), (0, DO - CLS)))
    bias2p = jnp.pad(bias2, (0, DO - CLS)).reshape(1, DO)
    bias1r = bias1.reshape(1, HID)

    W1f, W2f = _tc_weights(comp1, b1f, comp2, b2f, R)
    W1r = W1f.reshape(R, D, HID)
    W1dup = jnp.einsum('ab,rdj->adrbj', jnp.eye(2, dtype=x.dtype),
                       W1r).reshape(2 * D, R * 2 * HID)
    W2r = W2f.reshape(R, HID, DO)

    cnt2 = _sc_count(edge_index, et, N, R, E)
    winv16 = _tc_winv16(cnt2.reshape(NC, RN // 8, 8)).reshape(RN, 16)

    x2 = x.reshape(N // 2, 2 * D)
    y1 = _tc_y1(x2, W1dup, N, D, R, HID)
    acc1 = _sc_pass(edge_index, et, y1.reshape(RN, HID), winv16, N, R, E, HID)

    h, xr2 = _tc_h(x, root1, bias1r, acc1, root2p, bias2p, N, D, HID)
    y2 = _tc_y2(h, W2r, N, HID, R, DO)
    acc2 = _sc_pass(edge_index, et, y2, winv16, N, R, E, DO)

    out = _tc_logsoftmax(xr2, acc2, N, DO, CLS)
    return out[:, :CLS]


# speedup vs baseline: 1.2610x; 1.0589x over previous
import functools

import jax
import jax.numpy as jnp
from jax import lax
from jax.experimental import pallas as pl
from jax.experimental.pallas import tpu as pltpu
from jax.experimental.pallas import tpu_sc as plsc

NC = 2
NS = 16
NW = NC * NS
CH = 80
BN = 1000

_mesh = plsc.VectorSubcoreMesh(core_axis_name="c", subcore_axis_name="s")


def _sc_count(ei, et, N, R, E):
    RN = R * N
    CC = 400
    NSB = CC // CH
    EPW = E // NW
    NCH = EPW // CC
    ZB = RN // NS

    @functools.partial(
        pl.kernel,
        out_type=jax.ShapeDtypeStruct((NC * RN,), jnp.float32),
        mesh=_mesh,
        compiler_params=pltpu.CompilerParams(use_tc_tiling_on_sc=False),
        scratch_types=(
            [pltpu.VMEM_SHARED((RN,), jnp.float32)]
            + [pltpu.VMEM((CC,), jnp.int32) for _ in range(4)]
            + [pltpu.VMEM((CH,), jnp.int32) for _ in range(2 * NSB)]
            + [
                pltpu.VMEM((CH,), jnp.float32),
                pltpu.VMEM((CH,), jnp.float32),
                pltpu.VMEM((ZB,), jnp.float32),
                pltpu.SemaphoreType.DMA,
                pltpu.SemaphoreType.DMA,
                pltpu.SemaphoreType.DMA,
            ]
        ),
    )
    def k(ei_h, et_h, cnt_o, cnt_sh, d_v0, t_v0, d_v1, t_v1, *rest):
        iv_flat = rest[:2 * NSB]
        ones_v, zc_v, z_v, sem_e, sem_c0, sem_c1 = rest[2 * NSB:]
        cid = lax.axis_index("c")
        sid = lax.axis_index("s")
        wid = sid * NC + cid
        dv = (d_v0, d_v1)
        tv = (t_v0, t_v1)
        iv = (iv_flat[:NSB], iv_flat[NSB:])
        semc = (sem_c0, sem_c1)

        def zlp(j, c):
            z_v[pl.ds(j * 16, 16)] = jnp.zeros((16,), jnp.float32)
            return c

        lax.fori_loop(0, ZB // 16, zlp, 0)
        pltpu.sync_copy(z_v, cnt_sh.at[pl.ds(sid * ZB, ZB)])
        for j in range(CH // 16):
            s_ = pl.ds(j * 16, 16)
            ones_v[s_] = jnp.ones((16,), jnp.float32)
            zc_v[s_] = jnp.zeros((16,), jnp.float32)
        for p in range(2):
            for b in range(NSB):
                for j in range(CH // 16):
                    iv[p][b][pl.ds(j * 16, 16)] = jnp.zeros((16,), jnp.int32)
        for b in range(NSB):
            pltpu.async_copy(zc_v, cnt_sh.at[iv[0][b]], add=True, sem=sem_c0)
            pltpu.async_copy(zc_v, cnt_sh.at[iv[1][b]], add=True, sem=sem_c1)
        plsc.subcore_barrier()

        def load_edges(i, p):
            b = pl.multiple_of(wid * EPW + i * CC, 8)
            pltpu.async_copy(ei_h.at[1, pl.ds(b, CC)], dv[p], sem_e)
            pltpu.async_copy(et_h.at[pl.ds(b, CC)], tv[p], sem_e)

        def wait_edges(p):
            pltpu.make_async_copy(
                ei_h.at[1, pl.ds(0, CC)], dv[p], sem_e).wait()
            pltpu.make_async_copy(et_h.at[pl.ds(0, CC)], tv[p], sem_e).wait()

        def wait_scatters(p):
            for b in range(NSB):
                pltpu.make_async_copy(
                    ones_v, cnt_sh.at[iv[p][b]], semc[p]).wait()

        def step(i, p, po):
            wait_edges(p)
            wait_scatters(p)
            for b in range(NSB):
                for j in range(CH // 16):
                    s_ = pl.ds(j * 16, 16)
                    sc = pl.ds(b * CH + j * 16, 16)
                    iv[p][b][s_] = dv[p][sc] * R + tv[p][sc]

            @pl.when(i + 1 < NCH)
            def _():
                load_edges(i + 1, po)

            for b in range(NSB):
                pltpu.async_copy(
                    ones_v, cnt_sh.at[iv[p][b]], add=True, sem=semc[p])

        load_edges(0, 0)

        def body(kk, c):
            step(2 * kk, 0, 1)
            step(2 * kk + 1, 1, 0)
            return c

        lax.fori_loop(0, NCH // 2, body, 0)
        step(NCH - 1, 0, 1)
        wait_scatters(0)
        wait_scatters(1)
        plsc.subcore_barrier()
        pltpu.sync_copy(cnt_sh.at[pl.ds(sid * ZB, ZB)], z_v)
        pltpu.sync_copy(z_v, cnt_o.at[pl.ds(cid * RN + sid * ZB, ZB)])

    return k(ei, et)


def _sc_pass(ei, et, yf, w16, N, R, E, D):
    EPW = E // NW
    NCH = EPW // CH
    assert NCH % 2 == 1
    ZR = 80
    NZC = N // ZR
    NP = (NZC + NS - 1) // NS

    @functools.partial(
        pl.kernel,
        out_type=jax.ShapeDtypeStruct((NC, N, D), jnp.float32),
        mesh=_mesh,
        compiler_params=pltpu.CompilerParams(use_tc_tiling_on_sc=False),
        scratch_types=[
            pltpu.VMEM_SHARED((N, D), jnp.float32),
            pltpu.VMEM((CH,), jnp.int32),
            pltpu.VMEM((CH,), jnp.int32),
            pltpu.VMEM((CH,), jnp.int32),
            pltpu.VMEM((CH,), jnp.int32),
            pltpu.VMEM((CH,), jnp.int32),
            pltpu.VMEM((CH,), jnp.int32),
            pltpu.VMEM((CH,), jnp.int32),
            pltpu.VMEM((CH,), jnp.int32),
            pltpu.VMEM((CH,), jnp.int32),
            pltpu.VMEM((CH,), jnp.int32),
            pltpu.VMEM((CH, 16), jnp.float32),
            pltpu.VMEM((CH, 16), jnp.float32),
            pltpu.VMEM((CH, D), jnp.float32),
            pltpu.VMEM((CH, D), jnp.float32),
            pltpu.VMEM((ZR, D), jnp.float32),
            pltpu.VMEM((CH,), jnp.int32),
            pltpu.VMEM((CH,), jnp.int32),
            pltpu.VMEM((CH,), jnp.int32),
            pltpu.SemaphoreType.DMA,
            pltpu.SemaphoreType.DMA,
            pltpu.SemaphoreType.DMA,
            pltpu.SemaphoreType.DMA,
        ],
    )
    def k(ei_h, et_h, y_h, w16_h, acc_o, acc_sh,
          s_v0, d_v0, t_v0, g_v0, q_v0,
          s_v1, d_v1, t_v1, g_v1, q_v1,
          w_v0, w_v1, rows_v0, rows_v1, zr_v, si_v0, si_v1, izero,
          sem_e, sem_g, sem_sc0, sem_sc1):
        cid = lax.axis_index("c")
        sid = lax.axis_index("s")
        wid = sid * NC + cid
        sv = (s_v0, s_v1)
        dv = (d_v0, d_v1)
        tv = (t_v0, t_v1)
        gv = (g_v0, g_v1)
        qv = (q_v0, q_v1)
        wv = (w_v0, w_v1)
        rowsv = (rows_v0, rows_v1)
        siv = (si_v0, si_v1)
        semsc = (sem_sc0, sem_sc1)

        def zlp(i, c):
            for j in range(D // 16):
                zr_v[i, pl.ds(j * 16, 16)] = jnp.zeros((16,), jnp.float32)
            return c

        lax.fori_loop(0, ZR, zlp, 0)
        for p in range(NP):
            cidx = sid + p * NS

            @pl.when(cidx < NZC)
            def _():
                pltpu.sync_copy(zr_v, acc_sh.at[pl.ds(cidx * ZR, ZR)])

        plsc.subcore_barrier()

        def load_edges(i, p):
            b = pl.multiple_of(wid * EPW + i * CH, 8)
            pltpu.async_copy(ei_h.at[0, pl.ds(b, CH)], sv[p], sem_e)
            pltpu.async_copy(ei_h.at[1, pl.ds(b, CH)], dv[p], sem_e)
            pltpu.async_copy(et_h.at[pl.ds(b, CH)], tv[p], sem_e)

        def wait_edges(p):
            pltpu.make_async_copy(
                ei_h.at[0, pl.ds(0, CH)], sv[p], sem_e).wait()
            pltpu.make_async_copy(
                ei_h.at[1, pl.ds(0, CH)], dv[p], sem_e).wait()
            pltpu.make_async_copy(et_h.at[pl.ds(0, CH)], tv[p], sem_e).wait()

        def gq(p):
            for j in range(CH // 16):
                s_ = pl.ds(j * 16, 16)
                tt = tv[p][s_]
                gv[p][s_] = tt * N + sv[p][s_]
                qv[p][s_] = dv[p][s_] * R + tt
                siv[p][s_] = dv[p][s_]

        def issue_gathers(p):
            pltpu.async_copy(y_h.at[gv[p]], rowsv[p], sem_g)
            pltpu.async_copy(w16_h.at[qv[p]], wv[p], sem_g)

        def wait_gathers(p):
            pltpu.make_async_copy(y_h.at[gv[p]], rowsv[p], sem_g).wait()
            pltpu.make_async_copy(w16_h.at[qv[p]], wv[p], sem_g).wait()

        def scale(p):
            for e in range(CH):
                wsp = wv[p][e]
                for j in range(D // 16):
                    s_ = pl.ds(j * 16, 16)
                    rowsv[p][e, s_] = rowsv[p][e, s_] * wsp

        def issue_scatter(p):
            pltpu.async_copy(rowsv[p], acc_sh.at[siv[p]], add=True, sem=semsc[p])

        def wait_scatter(p):
            pltpu.make_async_copy(
                rowsv[p], acc_sh.at[siv[p]], semsc[p]).wait()

        for j in range(CH // 16):
            izero[pl.ds(j * 16, 16)] = jnp.zeros((16,), jnp.int32)
        pltpu.async_copy(zr_v, acc_sh.at[izero], add=True, sem=semsc[1])

        load_edges(0, 0)
        wait_edges(0)
        gq(0)
        issue_gathers(0)
        load_edges(1, 1)

        def body(kk, c):
            wait_edges(1)
            wait_scatter(1)
            gq(1)
            issue_gathers(1)
            wait_gathers(0)
            scale(0)
            issue_scatter(0)
            load_edges(2 * kk + 2, 0)
            wait_edges(0)
            wait_scatter(0)
            gq(0)
            issue_gathers(0)
            wait_gathers(1)
            scale(1)
            issue_scatter(1)

            @pl.when(2 * kk + 3 < NCH)
            def _():
                load_edges(2 * kk + 3, 1)

            return c

        lax.fori_loop(0, (NCH - 1) // 2, body, 0)
        wait_gathers(0)
        scale(0)
        pltpu.sync_copy(rowsv[0], acc_sh.at[siv[0]], add=True)
        wait_scatter(1)
        plsc.subcore_barrier()
        for p in range(NP):
            cidx = sid + p * NS

            @pl.when(cidx < NZC)
            def _():
                pltpu.sync_copy(acc_sh.at[pl.ds(cidx * ZR, ZR)], zr_v)
                pltpu.sync_copy(zr_v, acc_o.at[cid, pl.ds(cidx * ZR, ZR)])

    return k(ei, et, yf, w16)


def _tc_weights(comp1, b1f, comp2, b2f, R):
    def body(c1, b1, c2, b2, w1o, w2o):
        w1o[...] = jnp.dot(c1[...], b1[...], preferred_element_type=jnp.float32)
        w2o[...] = jnp.dot(c2[...], b2[...], preferred_element_type=jnp.float32)

    return pl.pallas_call(
        body,
        out_shape=(
            jax.ShapeDtypeStruct((R, b1f.shape[1]), jnp.float32),
            jax.ShapeDtypeStruct((R, b2f.shape[1]), jnp.float32),
        ),
    )(comp1, b1f, comp2, b2f)


def _tc_y1(x2, W1dup, N, D, R, HID):
    H2 = 2 * HID

    def body(x_ref, w_ref, o_ref):
        o_ref[...] = jnp.dot(x_ref[...], w_ref[...],
                             preferred_element_type=jnp.float32)

    return pl.pallas_call(
        body,
        grid=(R,),
        in_specs=[
            pl.BlockSpec((N // 2, 2 * D), lambda r: (0, 0)),
            pl.BlockSpec((2 * D, H2), lambda r: (0, r)),
        ],
        out_specs=pl.BlockSpec((N // 2, H2), lambda r: (r, 0)),
        out_shape=jax.ShapeDtypeStruct((R * N // 2, H2), jnp.float32),
    )(x2, W1dup)


def _tc_h(x, root1, bias1r, acc1, root2p, bias2r, N, D, HID):
    NT = N // BN

    def body(x_ref, r1_ref, b1_ref, a1_ref, r2_ref, b2_ref, h_ref, xr2_ref):
        h = jnp.dot(x_ref[...], r1_ref[...],
                    preferred_element_type=jnp.float32) + b1_ref[...]
        h = h + a1_ref[0] + a1_ref[1]
        h = jnp.maximum(h, 0.0)
        h_ref[...] = h
        xr2_ref[...] = jnp.dot(h, r2_ref[...],
                               preferred_element_type=jnp.float32) + b2_ref[...]

    return pl.pallas_call(
        body,
        grid=(NT,),
        in_specs=[
            pl.BlockSpec((BN, D), lambda nt: (nt, 0)),
            pl.BlockSpec((D, HID), lambda nt: (0, 0)),
            pl.BlockSpec((1, HID), lambda nt: (0, 0)),
            pl.BlockSpec((NC, BN, HID), lambda nt: (0, nt, 0)),
            pl.BlockSpec((HID, 128), lambda nt: (0, 0)),
            pl.BlockSpec((1, 128), lambda nt: (0, 0)),
        ],
        out_specs=(
            pl.BlockSpec((BN, HID), lambda nt: (nt, 0)),
            pl.BlockSpec((BN, 128), lambda nt: (nt, 0)),
        ),
        out_shape=(
            jax.ShapeDtypeStruct((N, HID), jnp.float32),
            jax.ShapeDtypeStruct((N, 128), jnp.float32),
        ),
    )(x, root1, bias1r, acc1, root2p, bias2r)


def _tc_y2(h, W2r, N, HID, R, DO):
    def body(h_ref, w_ref, o_ref):
        o_ref[...] = jnp.dot(h_ref[...], w_ref[0],
                             preferred_element_type=jnp.float32)

    return pl.pallas_call(
        body,
        grid=(R,),
        in_specs=[
            pl.BlockSpec((N, HID), lambda r: (0, 0)),
            pl.BlockSpec((1, HID, DO), lambda r: (r, 0, 0)),
        ],
        out_specs=pl.BlockSpec((N, DO), lambda r: (r, 0)),
        out_shape=jax.ShapeDtypeStruct((R * N, DO), jnp.float32),
    )(h, W2r)


def _tc_winv16(cnt2r):
    _, Q8, _ = cnt2r.shape
    NT = 10
    B8 = Q8 // NT

    def body(c_ref, o_ref):
        c = c_ref[0] + c_ref[1]
        w = 1.0 / jnp.maximum(c, 1.0)
        k = lax.broadcasted_iota(jnp.int32, (8, 128), 0)
        cc = lax.broadcasted_iota(jnp.int32, (8, 128), 1)
        sel = (cc // 16 == k).astype(jnp.float32)
        o_ref[...] = jnp.dot(w, sel, preferred_element_type=jnp.float32)

    return pl.pallas_call(
        body,
        grid=(NT,),
        in_specs=[pl.BlockSpec((2, B8, 8), lambda i: (0, i, 0))],
        out_specs=pl.BlockSpec((B8, 128), lambda i: (i, 0)),
        out_shape=jax.ShapeDtypeStruct((Q8, 128), jnp.float32),
    )(cnt2r)


def _tc_logsoftmax(xr2, acc2, N, DO, CLS):
    NT = N // BN

    def body(xr_ref, a_ref, o_ref):
        z = xr_ref[...] + a_ref[0] + a_ref[1]
        col = lax.broadcasted_iota(jnp.int32, z.shape, 1)
        z = jnp.where(col < CLS, z, -1e30)
        m = jnp.max(z, axis=1, keepdims=True)
        e = jnp.exp(z - m)
        s = jnp.sum(e, axis=1, keepdims=True)
        o_ref[...] = z - m - jnp.log(s)

    return pl.pallas_call(
        body,
        grid=(NT,),
        in_specs=[
            pl.BlockSpec((BN, DO), lambda nt: (nt, 0)),
            pl.BlockSpec((NC, BN, DO), lambda nt: (0, nt, 0)),
        ],
        out_specs=pl.BlockSpec((BN, DO), lambda nt: (nt, 0)),
        out_shape=jax.ShapeDtypeStruct((N, DO), jnp.float32),
    )(xr2, acc2)


def kernel(x, edge_index, edge_type, basis1, comp1, root1, bias1,
           basis2, comp2, root2, bias2):
    N, D = x.shape
    HID = root1.shape[1]
    CLS = root2.shape[1]
    R = comp1.shape[0]
    NB = basis1.shape[0]
    E = edge_type.shape[0]
    DO = 128
    RN = R * N

    et = edge_type

    b1f = basis1.reshape(NB, D * HID)
    b2p = jnp.pad(basis2, ((0, 0), (0, 0), (0, DO - CLS)))
    b2f = b2p.reshape(NB, HID * DO)
    root2p = jnp.pad(root2, ((0, 0), (0, DO - CLS)))
    bias2p = jnp.pad(bias2, (0, DO - CLS)).reshape(1, DO)
    bias1r = bias1.reshape(1, HID)

    W1f, W2f = _tc_weights(comp1, b1f, comp2, b2f, R)
    W1r = W1f.reshape(R, D, HID)
    W1dup = jnp.einsum('ab,rdj->adrbj', jnp.eye(2, dtype=x.dtype),
                       W1r).reshape(2 * D, R * 2 * HID)
    W2r = W2f.reshape(R, HID, DO)

    cnt2 = _sc_count(edge_index, et, N, R, E)
    winv16 = _tc_winv16(cnt2.reshape(NC, RN // 8, 8)).reshape(RN, 16)

    x2 = x.reshape(N // 2, 2 * D)
    y1 = _tc_y1(x2, W1dup, N, D, R, HID)
    acc1 = _sc_pass(edge_index, et, y1.reshape(RN, HID), winv16, N, R, E, HID)

    h, xr2 = _tc_h(x, root1, bias1r, acc1, root2p, bias2p, N, D, HID)
    y2 = _tc_y2(h, W2r, N, HID, R, DO)
    acc2 = _sc_pass(edge_index, et, y2, winv16, N, R, E, DO)

    out = _tc_logsoftmax(xr2, acc2, N, DO, CLS)
    return out[:, :CLS]
